# baseline (speedup 1.0000x reference)
import functools

import jax
import jax.numpy as jnp
from jax import lax
from jax.experimental import pallas as pl
from jax.experimental.pallas import tpu as pltpu
from jax.experimental.pallas import tpu_sc as plsc

N = 10000
D_IN = 128
D = 256
E = 320000

NC = 2
NS = 16
NW = NC * NS
BKT = 320
TRASH = BKT
STR = D + 16
CAP = 480
NSEG = NW * NW * CAP

E_T1 = E // NW
SC1 = 2000
NSC1 = E_T1 // SC1
CH = 80
NCH1 = SC1 // CH

MGC = 52429
SHF = 24


@functools.cache
def _build_p1():
  mesh = plsc.VectorSubcoreMesh(
      core_axis_name="c", subcore_axis_name="s", num_cores=NC, num_subcores=NS)

  @functools.partial(
      pl.kernel,
      out_type=(
          jax.ShapeDtypeStruct((NSEG,), jnp.int32),
          jax.ShapeDtypeStruct((NSEG,), jnp.int32),
          jax.ShapeDtypeStruct((NSEG,), jnp.float32),
          jax.ShapeDtypeStruct((NW * NW,), jnp.int32),
      ),
      mesh=mesh,
      scratch_types=[
          pltpu.VMEM((SC1,), jnp.int32),
          pltpu.VMEM((SC1,), jnp.int32),
          pltpu.VMEM((SC1,), jnp.float32),
          pltpu.VMEM((SC1,), jnp.int32),
          pltpu.VMEM((NCH1, CH), jnp.int32),
          pltpu.VMEM((NW,), jnp.int32),
          pltpu.SMEM((NW,), jnp.int32),
          pltpu.SemaphoreType.DMA,
      ],
  )
  def _p1(row_hbm, col_hbm, attr_hbm, loc_out, col_out, attr_out, cnt_out,
          row_v, col_v, attr_v, locbuf, pos2d, cntbuf, ctr_s, sem):
    c = lax.axis_index("c")
    s = lax.axis_index("s")
    w = s * NC + c
    e0 = w * E_T1
    iota = lax.iota(jnp.int32, 16)
    for b in range(NW):
      ctr_s[b] = 0

    def _super(m, _):
      es = e0 + m * SC1
      pltpu.sync_copy(row_hbm.at[pl.ds(es, SC1)], row_v)
      pltpu.sync_copy(col_hbm.at[pl.ds(es, SC1)], col_v)
      pltpu.sync_copy(attr_hbm.at[pl.ds(es, SC1)], attr_v)

      def _chunk(k, _):
        cb = k * CH
        for j in range(CH // 16):
          r16 = row_v[pl.ds(cb + j * 16, 16)]
          b16 = (r16 * MGC) >> SHF
          loc16 = r16 - b16 * BKT
          pos16 = iota * 0
          for l in range(16):
            b = b16[l]
            p = jnp.minimum(ctr_s[b], CAP - 1)
            ctr_s[b] = p + 1
            pos16 = jnp.where(iota == l, (b * NW + w) * CAP + p, pos16)
          locbuf[pl.ds(cb + j * 16, 16)] = loc16
          pos2d[k, pl.ds(j * 16, 16)] = pos16
        idx = pos2d.at[k]
        pltpu.async_copy(locbuf.at[pl.ds(cb, CH)], loc_out.at[idx], sem).wait()
        pltpu.async_copy(col_v.at[pl.ds(cb, CH)], col_out.at[idx], sem).wait()
        pltpu.async_copy(attr_v.at[pl.ds(cb, CH)], attr_out.at[idx], sem).wait()
        return 0

      lax.fori_loop(0, NCH1, _chunk, 0)
      return 0

    lax.fori_loop(0, NSC1, _super, 0)

    v0 = iota * 0
    v1 = iota * 0
    for l in range(16):
      v0 = jnp.where(iota == l, ctr_s[l], v0)
      v1 = jnp.where(iota == l, ctr_s[16 + l], v1)
    cntbuf[pl.ds(0, 16)] = v0
    cntbuf[pl.ds(16, 16)] = v1
    pltpu.sync_copy(cntbuf, cnt_out.at[pl.ds(w * NW, NW)])

  return _p1


ACC_W = (BKT + 8) * STR
NZACC = ACC_W // 16


@functools.cache
def _build_p2():
  mesh = plsc.VectorSubcoreMesh(
      core_axis_name="c", subcore_axis_name="s", num_cores=NC, num_subcores=NS)

  @functools.partial(
      pl.kernel,
      out_type=jax.ShapeDtypeStruct((NW * BKT, D), jnp.float32),
      mesh=mesh,
      scratch_types=[
          pltpu.VMEM((ACC_W,), jnp.float32),
          pltpu.VMEM((CH, D), jnp.float32),
          pltpu.VMEM((CAP,), jnp.int32),
          pltpu.VMEM((CAP,), jnp.int32),
          pltpu.VMEM((CAP,), jnp.float32),
          pltpu.VMEM((NW * NW,), jnp.int32),
          pltpu.SemaphoreType.DMA,
      ],
  )
  def _p2(h_hbm, loc_hbm, col_hbm, attr_hbm, cnt_hbm, accn_out,
          acc_v, gbuf, segl, segc, sega, cnt_v, sem):
    c = lax.axis_index("c")
    s = lax.axis_index("s")
    w = s * NC + c
    iota = lax.iota(jnp.int32, 16)
    zero16 = jnp.zeros((16,), jnp.float32)
    fzero = jnp.zeros((16,), jnp.float32)

    def _zacc(i, _):
      acc_v[pl.ds(i * 16, 16)] = zero16
      return 0

    lax.fori_loop(0, NZACC, _zacc, 0)
    pltpu.sync_copy(cnt_hbm, cnt_v)

    def _src(src, _):
      off = src * NW + (w & 16)
      vv = cnt_v[pl.ds(off, 16)]
      iw = (iota + (w & 15)) & 15
      g = lax.gather(
          vv, iw[:, None],
          lax.GatherDimensionNumbers(offset_dims=(), collapsed_slice_dims=(0,),
                                     start_index_map=(0,)),
          (1,), mode=lax.GatherScatterMode.PROMISE_IN_BOUNDS)
      cnt = g[0]
      segbase = (w * NW + src) * CAP
      pltpu.sync_copy(loc_hbm.at[pl.ds(segbase, CAP)], segl)
      pltpu.sync_copy(col_hbm.at[pl.ds(segbase, CAP)], segc)
      pltpu.sync_copy(attr_hbm.at[pl.ds(segbase, CAP)], sega)
      for jj in range(CAP // 16):
        sl = pl.ds(jj * 16, 16)
        valid = (jj * 16 + iota) < cnt
        segc[sl] = jnp.where(valid, segc[sl], 0)
        segl[sl] = jnp.where(valid, segl[sl], TRASH)
      nch = ((cnt + (CH - 1)) * 52429) >> 22

      def _chunk(k, _):
        cb = k * CH
        pltpu.async_copy(h_hbm.at[segc.at[pl.ds(cb, CH)]], gbuf, sem).wait()
        for jj in range(CH // 16):
          l16 = segl[pl.ds(cb + jj * 16, 16)]
          a16 = sega[pl.ds(cb + jj * 16, 16)]
          for l in range(16):
            loc = l16[l]
            a = a16[l]
            fb = loc * STR
            gi = jj * 16 + l
            for j in range(D // 16):
              v = gbuf[gi, pl.ds(j * 16, 16)] * a
              plsc.addupdate(acc_v.at[pl.ds(fb + j * 16, 16)], v)
            av = jnp.where(iota == 0, a, fzero)
            plsc.addupdate(acc_v.at[pl.ds(fb + D, 16)], av)
        return 0

      lax.fori_loop(0, nch, _chunk, 0)
      return 0

    lax.fori_loop(0, NW, _src, 0)

    for q in range(BKT // CH):
      def _norm(i, _):
        fb = (q * CH + i) * STR
        rs16 = acc_v[pl.ds(fb + D, 16)]
        inv16 = 1.0 / (rs16 + 1e-9)
        inv = inv16[0]
        for j in range(D // 16):
          gbuf[i, pl.ds(j * 16, 16)] = acc_v[pl.ds(fb + j * 16, 16)] * inv
        return 0

      lax.fori_loop(0, CH, _norm, 0)
      pltpu.sync_copy(gbuf, accn_out.at[pl.ds(w * BKT + q * CH, CH)])

  return _p2


def _mm_body(x_ref, w_ref, o_ref):
    o_ref[...] = jnp.dot(x_ref[...], w_ref[...],
                         preferred_element_type=jnp.float32)


def _epi_body(acc_ref, b0_ref, fc_ref, bf_ref, o_ref):
    vh = acc_ref[...] + b0_ref[...]
    t = jnp.sum(vh * fc_ref[...], axis=1, keepdims=True) + bf_ref[...]
    g = jax.nn.sigmoid(t)
    zero = jnp.zeros_like(vh)
    o_ref[...] = jnp.where(vh < 0, zero, vh) + g * jnp.where(vh > 0, zero, vh)


def kernel(x, edge_index, edge_attr, W0, b0, fc0, bf0):
    h = pl.pallas_call(
        _mm_body,
        grid=(10,),
        in_specs=[
            pl.BlockSpec((N // 10, D_IN), lambda i: (i, 0)),
            pl.BlockSpec((D_IN, D), lambda i: (0, 0)),
        ],
        out_specs=pl.BlockSpec((N // 10, D), lambda i: (i, 0)),
        out_shape=jax.ShapeDtypeStruct((N, D), jnp.float32),
    )(x, W0)

    row = edge_index[0]
    col = edge_index[1]

    loc_a, col_a, attr_a, counts = _build_p1()(row, col, edge_attr)

    accn = _build_p2()(h, loc_a, col_a, attr_a, counts)

    out = pl.pallas_call(
        _epi_body,
        grid=(10,),
        in_specs=[
            pl.BlockSpec((N // 10, D), lambda i: (i, 0)),
            pl.BlockSpec((1, D), lambda i: (0, 0)),
            pl.BlockSpec((1, D), lambda i: (0, 0)),
            pl.BlockSpec((1, 1), lambda i: (0, 0)),
        ],
        out_specs=pl.BlockSpec((N // 10, D), lambda i: (i, 0)),
        out_shape=jax.ShapeDtypeStruct((N, D), jnp.float32),
    )(accn, b0.reshape(1, D), fc0.reshape(1, D), bf0.reshape(1, 1))
    return out


# speedup vs baseline: 1.1099x; 1.1099x over previous
import functools

import jax
import jax.numpy as jnp
from jax import lax
from jax.experimental import pallas as pl
from jax.experimental.pallas import tpu as pltpu
from jax.experimental.pallas import tpu_sc as plsc

N = 10000
D_IN = 128
D = 256
E = 320000

NC = 2
NS = 16
NW = NC * NS
BKT = 320
TRASH = BKT
STR = D + 16
CAP = 480
NSEG = NW * NW * CAP

E_T1 = E // NW
SC1 = 2000
NSC1 = E_T1 // SC1
CH = 80
NCH1 = SC1 // CH

MGC = 52429
SHF = 24


@functools.cache
def _build_p1():
  mesh = plsc.VectorSubcoreMesh(
      core_axis_name="c", subcore_axis_name="s", num_cores=NC, num_subcores=NS)

  @functools.partial(
      pl.kernel,
      out_type=(
          jax.ShapeDtypeStruct((NSEG,), jnp.int32),
          jax.ShapeDtypeStruct((NSEG,), jnp.int32),
          jax.ShapeDtypeStruct((NSEG,), jnp.float32),
          jax.ShapeDtypeStruct((NW * NW,), jnp.int32),
      ),
      mesh=mesh,
      scratch_types=[
          pltpu.VMEM((SC1,), jnp.int32),
          pltpu.VMEM((SC1,), jnp.int32),
          pltpu.VMEM((SC1,), jnp.float32),
          pltpu.VMEM((SC1,), jnp.int32),
          pltpu.VMEM((NCH1, CH), jnp.int32),
          pltpu.VMEM((NW,), jnp.int32),
          pltpu.SMEM((NW,), jnp.int32),
          pltpu.SemaphoreType.DMA,
      ],
  )
  def _p1(row_hbm, col_hbm, attr_hbm, loc_out, col_out, attr_out, cnt_out,
          row_v, col_v, attr_v, locbuf, pos2d, cntbuf, ctr_s, sem):
    c = lax.axis_index("c")
    s = lax.axis_index("s")
    w = s * NC + c
    e0 = w * E_T1
    iota = lax.iota(jnp.int32, 16)
    for b in range(NW):
      ctr_s[b] = 0

    def _super(m, _):
      es = e0 + m * SC1
      pltpu.sync_copy(row_hbm.at[pl.ds(es, SC1)], row_v)
      pltpu.sync_copy(col_hbm.at[pl.ds(es, SC1)], col_v)
      pltpu.sync_copy(attr_hbm.at[pl.ds(es, SC1)], attr_v)

      def _chunk(k, _):
        cb = k * CH

        @pl.when(k > 0)
        def _drain_prev():
          cbp = cb - CH
          idxp = pos2d.at[k - 1]
          pltpu.make_async_copy(
              locbuf.at[pl.ds(cbp, CH)], loc_out.at[idxp], sem).wait()
          pltpu.make_async_copy(
              col_v.at[pl.ds(cbp, CH)], col_out.at[idxp], sem).wait()
          pltpu.make_async_copy(
              attr_v.at[pl.ds(cbp, CH)], attr_out.at[idxp], sem).wait()

        for j in range(CH // 16):
          r16 = row_v[pl.ds(cb + j * 16, 16)]
          b16 = (r16 * MGC) >> SHF
          loc16 = r16 - b16 * BKT
          pos16 = iota * 0
          for l in range(16):
            b = b16[l]
            p = jnp.minimum(ctr_s[b], CAP - 1)
            ctr_s[b] = p + 1
            pos16 = jnp.where(iota == l, (b * NW + w) * CAP + p, pos16)
          locbuf[pl.ds(cb + j * 16, 16)] = loc16
          pos2d[k, pl.ds(j * 16, 16)] = pos16
        idx = pos2d.at[k]
        pltpu.async_copy(locbuf.at[pl.ds(cb, CH)], loc_out.at[idx], sem)
        pltpu.async_copy(col_v.at[pl.ds(cb, CH)], col_out.at[idx], sem)
        pltpu.async_copy(attr_v.at[pl.ds(cb, CH)], attr_out.at[idx], sem)
        return 0

      lax.fori_loop(0, NCH1, _chunk, 0)
      cbl = SC1 - CH
      idxl = pos2d.at[NCH1 - 1]
      pltpu.make_async_copy(
          locbuf.at[pl.ds(cbl, CH)], loc_out.at[idxl], sem).wait()
      pltpu.make_async_copy(
          col_v.at[pl.ds(cbl, CH)], col_out.at[idxl], sem).wait()
      pltpu.make_async_copy(
          attr_v.at[pl.ds(cbl, CH)], attr_out.at[idxl], sem).wait()
      return 0

    lax.fori_loop(0, NSC1, _super, 0)

    v0 = iota * 0
    v1 = iota * 0
    for l in range(16):
      v0 = jnp.where(iota == l, ctr_s[l], v0)
      v1 = jnp.where(iota == l, ctr_s[16 + l], v1)
    cntbuf[pl.ds(0, 16)] = v0
    cntbuf[pl.ds(16, 16)] = v1
    pltpu.sync_copy(cntbuf, cnt_out.at[pl.ds(w * NW, NW)])

  return _p1


ACC_W = (BKT + 8) * STR
NZACC = ACC_W // 16
CH2 = 64


@functools.cache
def _build_p2():
  mesh = plsc.VectorSubcoreMesh(
      core_axis_name="c", subcore_axis_name="s", num_cores=NC, num_subcores=NS)

  @functools.partial(
      pl.kernel,
      out_type=jax.ShapeDtypeStruct((NW * BKT, D), jnp.float32),
      mesh=mesh,
      scratch_types=[
          pltpu.VMEM((ACC_W,), jnp.float32),
          pltpu.VMEM((2 * CH2, D), jnp.float32),
          pltpu.VMEM((CAP,), jnp.int32),
          pltpu.VMEM((CAP,), jnp.int32),
          pltpu.VMEM((CAP,), jnp.float32),
          pltpu.VMEM((NW * NW,), jnp.int32),
          pltpu.SemaphoreType.DMA,
          pltpu.SemaphoreType.DMA,
          pltpu.SemaphoreType.DMA,
      ],
  )
  def _p2(h_hbm, loc_hbm, col_hbm, attr_hbm, cnt_hbm, accn_out,
          acc_v, gbuf, segl, segc, sega, cnt_v, sem_a, sem_b, sem_s):
    c = lax.axis_index("c")
    s = lax.axis_index("s")
    w = s * NC + c
    iota = lax.iota(jnp.int32, 16)
    zero16 = jnp.zeros((16,), jnp.float32)
    fzero = jnp.zeros((16,), jnp.float32)

    def _zacc(i, _):
      acc_v[pl.ds(i * 16, 16)] = zero16
      return 0

    lax.fori_loop(0, NZACC, _zacc, 0)
    pltpu.sync_copy(cnt_hbm, cnt_v)

    def _do_chunk(cb, h0):
      for jj in range(CH2 // 16):
        l16 = segl[pl.ds(cb + jj * 16, 16)]
        a16 = sega[pl.ds(cb + jj * 16, 16)]
        for l in range(16):
          loc = l16[l]
          a = a16[l]
          fb = loc * STR
          gi = h0 + jj * 16 + l
          for j in range(D // 16):
            v = gbuf[gi, pl.ds(j * 16, 16)] * a
            plsc.addupdate(acc_v.at[pl.ds(fb + j * 16, 16)], v)
          av = jnp.where(iota == 0, a, fzero)
          plsc.addupdate(acc_v.at[pl.ds(fb + D, 16)], av)

    def _issue(k, h0, sem):
      pltpu.async_copy(h_hbm.at[segc.at[pl.ds(k * CH2, CH2)]],
                       gbuf.at[pl.ds(h0, CH2)], sem)

    def _wait(k, h0, sem):
      pltpu.make_async_copy(h_hbm.at[segc.at[pl.ds(k * CH2, CH2)]],
                            gbuf.at[pl.ds(h0, CH2)], sem).wait()

    def _src(src, _):
      off = src * NW + (w & 16)
      vv = cnt_v[pl.ds(off, 16)]
      iw = (iota + (w & 15)) & 15
      g = lax.gather(
          vv, iw[:, None],
          lax.GatherDimensionNumbers(offset_dims=(), collapsed_slice_dims=(0,),
                                     start_index_map=(0,)),
          (1,), mode=lax.GatherScatterMode.PROMISE_IN_BOUNDS)
      cnt = g[0]
      segbase = (w * NW + src) * CAP
      d1 = pltpu.async_copy(loc_hbm.at[pl.ds(segbase, CAP)], segl, sem_s)
      d2 = pltpu.async_copy(col_hbm.at[pl.ds(segbase, CAP)], segc, sem_s)
      d3 = pltpu.async_copy(attr_hbm.at[pl.ds(segbase, CAP)], sega, sem_s)
      d1.wait()
      d2.wait()
      d3.wait()
      for jj in range(CAP // 16):
        sl = pl.ds(jj * 16, 16)
        valid = (jj * 16 + iota) < cnt
        segc[sl] = jnp.where(valid, segc[sl], 0)
        segl[sl] = jnp.where(valid, segl[sl], TRASH)
      nch = (cnt + (CH2 - 1)) >> 6

      @pl.when(nch > 0)
      def _prime():
        _issue(0, 0, sem_a)

      def _pair(kk, _):
        k0 = 2 * kk
        k1 = k0 + 1

        @pl.when(k1 < nch)
        def _issue_odd():
          _issue(k1, CH2, sem_b)

        _wait(k0, 0, sem_a)
        _do_chunk(k0 * CH2, 0)

        @pl.when(k1 < nch)
        def _odd_chunk():
          @pl.when(k1 + 1 < nch)
          def _issue_next_even():
            _issue(k1 + 1, 0, sem_a)

          _wait(k1, CH2, sem_b)
          _do_chunk(k1 * CH2, CH2)

        return 0

      lax.fori_loop(0, (nch + 1) >> 1, _pair, 0)
      return 0

    lax.fori_loop(0, NW, _src, 0)

    for q in range(BKT // CH2):
      def _norm(i, _):
        fb = (q * CH2 + i) * STR
        rs16 = acc_v[pl.ds(fb + D, 16)]
        inv16 = 1.0 / (rs16 + 1e-9)
        inv = inv16[0]
        for j in range(D // 16):
          gbuf[i, pl.ds(j * 16, 16)] = acc_v[pl.ds(fb + j * 16, 16)] * inv
        return 0

      lax.fori_loop(0, CH2, _norm, 0)
      pltpu.sync_copy(gbuf.at[pl.ds(0, CH2)],
                      accn_out.at[pl.ds(w * BKT + q * CH2, CH2)])

  return _p2


def _mm_body(x_ref, w_ref, o_ref):
    o_ref[...] = jnp.dot(x_ref[...], w_ref[...],
                         preferred_element_type=jnp.float32)


def _epi_body(acc_ref, b0_ref, fc_ref, bf_ref, o_ref):
    vh = acc_ref[...] + b0_ref[...]
    t = jnp.sum(vh * fc_ref[...], axis=1, keepdims=True) + bf_ref[...]
    g = jax.nn.sigmoid(t)
    zero = jnp.zeros_like(vh)
    o_ref[...] = jnp.where(vh < 0, zero, vh) + g * jnp.where(vh > 0, zero, vh)


def kernel(x, edge_index, edge_attr, W0, b0, fc0, bf0):
    h = pl.pallas_call(
        _mm_body,
        grid=(10,),
        in_specs=[
            pl.BlockSpec((N // 10, D_IN), lambda i: (i, 0)),
            pl.BlockSpec((D_IN, D), lambda i: (0, 0)),
        ],
        out_specs=pl.BlockSpec((N // 10, D), lambda i: (i, 0)),
        out_shape=jax.ShapeDtypeStruct((N, D), jnp.float32),
    )(x, W0)

    row = edge_index[0]
    col = edge_index[1]

    loc_a, col_a, attr_a, counts = _build_p1()(row, col, edge_attr)

    accn = _build_p2()(h, loc_a, col_a, attr_a, counts)

    out = pl.pallas_call(
        _epi_body,
        grid=(10,),
        in_specs=[
            pl.BlockSpec((N // 10, D), lambda i: (i, 0)),
            pl.BlockSpec((1, D), lambda i: (0, 0)),
            pl.BlockSpec((1, D), lambda i: (0, 0)),
            pl.BlockSpec((1, 1), lambda i: (0, 0)),
        ],
        out_specs=pl.BlockSpec((N // 10, D), lambda i: (i, 0)),
        out_shape=jax.ShapeDtypeStruct((N, D), jnp.float32),
    )(accn, b0.reshape(1, D), fc0.reshape(1, D), bf0.reshape(1, 1))
    return out


# speedup vs baseline: 1.3819x; 1.2451x over previous
import functools

import jax
import jax.numpy as jnp
from jax import lax
from jax.experimental import pallas as pl
from jax.experimental.pallas import tpu as pltpu
from jax.experimental.pallas import tpu_sc as plsc

N = 10000
D_IN = 128
D = 256
E = 320000

NC = 2
NS = 16
NW = NC * NS
BKT = 320
TRASH = BKT
STR = D + 16
CAP = 480
NSEG = NW * NW * CAP

E_T1 = E // NW
SC1 = 2000
NSC1 = E_T1 // SC1
CH = 80
NCH1 = SC1 // CH

MGC = 52429
SHF = 24


@functools.cache
def _build_p1():
  mesh = plsc.VectorSubcoreMesh(
      core_axis_name="c", subcore_axis_name="s", num_cores=NC, num_subcores=NS)

  @functools.partial(
      pl.kernel,
      out_type=(
          jax.ShapeDtypeStruct((NSEG,), jnp.int32),
          jax.ShapeDtypeStruct((NSEG,), jnp.float32),
          jax.ShapeDtypeStruct((NW * NW,), jnp.int32),
      ),
      mesh=mesh,
      scratch_types=[
          pltpu.VMEM((SC1,), jnp.int32),
          pltpu.VMEM((SC1,), jnp.int32),
          pltpu.VMEM((SC1,), jnp.float32),
          pltpu.VMEM((SC1,), jnp.int32),
          pltpu.VMEM((NCH1, CH), jnp.int32),
          pltpu.VMEM((NW,), jnp.int32),
          pltpu.SMEM((NW,), jnp.int32),
          pltpu.SemaphoreType.DMA,
      ],
  )
  def _p1(row_hbm, col_hbm, attr_hbm, pk_out, attr_out, cnt_out,
          row_v, col_v, attr_v, locbuf, pos2d, cntbuf, ctr_s, sem):
    c = lax.axis_index("c")
    s = lax.axis_index("s")
    w = s * NC + c
    e0 = w * E_T1
    iota = lax.iota(jnp.int32, 16)
    for b in range(NW):
      ctr_s[b] = 0

    def _super(m, _):
      es = e0 + m * SC1
      pltpu.sync_copy(row_hbm.at[pl.ds(es, SC1)], row_v)
      pltpu.sync_copy(col_hbm.at[pl.ds(es, SC1)], col_v)
      pltpu.sync_copy(attr_hbm.at[pl.ds(es, SC1)], attr_v)

      def _chunk(k, _):
        cb = k * CH

        @pl.when(k > 0)
        def _drain_prev():
          cbp = cb - CH
          idxp = pos2d.at[k - 1]
          pltpu.make_async_copy(
              locbuf.at[pl.ds(cbp, CH)], pk_out.at[idxp], sem).wait()
          pltpu.make_async_copy(
              attr_v.at[pl.ds(cbp, CH)], attr_out.at[idxp], sem).wait()

        for j in range(CH // 16):
          r16 = row_v[pl.ds(cb + j * 16, 16)]
          c16 = col_v[pl.ds(cb + j * 16, 16)]
          b16 = (r16 * MGC) >> SHF
          loc16 = r16 - b16 * BKT
          pos16 = iota * 0
          for l in range(16):
            b = b16[l]
            p = jnp.minimum(ctr_s[b], CAP - 1)
            ctr_s[b] = p + 1
            pos16 = jnp.where(iota == l, (b * NW + w) * CAP + p, pos16)
          locbuf[pl.ds(cb + j * 16, 16)] = loc16 * 16384 + c16
          pos2d[k, pl.ds(j * 16, 16)] = pos16
        idx = pos2d.at[k]
        pltpu.async_copy(locbuf.at[pl.ds(cb, CH)], pk_out.at[idx], sem)
        pltpu.async_copy(attr_v.at[pl.ds(cb, CH)], attr_out.at[idx], sem)
        return 0

      lax.fori_loop(0, NCH1, _chunk, 0)
      cbl = SC1 - CH
      idxl = pos2d.at[NCH1 - 1]
      pltpu.make_async_copy(
          locbuf.at[pl.ds(cbl, CH)], pk_out.at[idxl], sem).wait()
      pltpu.make_async_copy(
          attr_v.at[pl.ds(cbl, CH)], attr_out.at[idxl], sem).wait()
      return 0

    lax.fori_loop(0, NSC1, _super, 0)

    v0 = iota * 0
    v1 = iota * 0
    for l in range(16):
      v0 = jnp.where(iota == l, ctr_s[l], v0)
      v1 = jnp.where(iota == l, ctr_s[16 + l], v1)
    cntbuf[pl.ds(0, 16)] = v0
    cntbuf[pl.ds(16, 16)] = v1
    pltpu.sync_copy(cntbuf, cnt_out.at[pl.ds(w * NW, NW)])

  return _p1


ACC_W = (BKT + 8) * STR
NZACC = ACC_W // 16
CH2 = 64


@functools.cache
def _build_p2():
  mesh = plsc.VectorSubcoreMesh(
      core_axis_name="c", subcore_axis_name="s", num_cores=NC, num_subcores=NS)

  @functools.partial(
      pl.kernel,
      out_type=jax.ShapeDtypeStruct((NW * BKT, D), jnp.float32),
      mesh=mesh,
      scratch_types=[
          pltpu.VMEM((ACC_W,), jnp.float32),
          pltpu.VMEM((2 * CH2, D), jnp.float32),
          pltpu.VMEM((CAP,), jnp.int32),
          pltpu.VMEM((CAP,), jnp.int32),
          pltpu.VMEM((CAP,), jnp.float32),
          pltpu.VMEM((NW * NW,), jnp.int32),
          pltpu.SemaphoreType.DMA,
          pltpu.SemaphoreType.DMA,
          pltpu.SemaphoreType.DMA,
      ],
  )
  def _p2(h_hbm, pk_hbm, attr_hbm, cnt_hbm, accn_out,
          acc_v, gbuf, segl, segc, sega, cnt_v, sem_a, sem_b, sem_s):
    c = lax.axis_index("c")
    s = lax.axis_index("s")
    w = s * NC + c
    iota = lax.iota(jnp.int32, 16)
    zero16 = jnp.zeros((16,), jnp.float32)
    fzero = jnp.zeros((16,), jnp.float32)

    def _zacc(i, _):
      acc_v[pl.ds(i * 16, 16)] = zero16
      return 0

    lax.fori_loop(0, NZACC, _zacc, 0)
    pltpu.sync_copy(cnt_hbm, cnt_v)

    def _do_chunk(cb, h0):
      for jj in range(CH2 // 16):
        l16 = segl[pl.ds(cb + jj * 16, 16)]
        a16 = sega[pl.ds(cb + jj * 16, 16)]
        for l in range(16):
          loc = l16[l]
          a = a16[l]
          fb = loc * STR
          gi = h0 + jj * 16 + l
          for j0 in range(0, D // 16, 8):
            vs = [gbuf[gi, pl.ds((j0 + j) * 16, 16)] * a for j in range(8)]
            for j in range(8):
              plsc.addupdate(acc_v.at[pl.ds(fb + (j0 + j) * 16, 16)], vs[j])
          av = jnp.where(iota == 0, a, fzero)
          plsc.addupdate(acc_v.at[pl.ds(fb + D, 16)], av)

    def _issue(k, h0, sem):
      pltpu.async_copy(h_hbm.at[segc.at[pl.ds(k * CH2, CH2)]],
                       gbuf.at[pl.ds(h0, CH2)], sem)

    def _wait(k, h0, sem):
      pltpu.make_async_copy(h_hbm.at[segc.at[pl.ds(k * CH2, CH2)]],
                            gbuf.at[pl.ds(h0, CH2)], sem).wait()

    def _src(src, _):
      off = src * NW + (w & 16)
      vv = cnt_v[pl.ds(off, 16)]
      iw = (iota + (w & 15)) & 15
      g = lax.gather(
          vv, iw[:, None],
          lax.GatherDimensionNumbers(offset_dims=(), collapsed_slice_dims=(0,),
                                     start_index_map=(0,)),
          (1,), mode=lax.GatherScatterMode.PROMISE_IN_BOUNDS)
      cnt = g[0]
      segbase = (w * NW + src) * CAP
      d1 = pltpu.async_copy(pk_hbm.at[pl.ds(segbase, CAP)], segl, sem_s)
      d2 = pltpu.async_copy(attr_hbm.at[pl.ds(segbase, CAP)], sega, sem_s)
      d1.wait()
      d2.wait()
      for jj in range(CAP // 16):
        sl = pl.ds(jj * 16, 16)
        valid = (jj * 16 + iota) < cnt
        pk = jnp.where(valid, segl[sl], TRASH * 16384)
        segc[sl] = pk & 16383
        segl[sl] = lax.shift_right_logical(pk, 14)
      nch = (cnt + (CH2 - 1)) >> 6

      @pl.when(nch > 0)
      def _prime():
        _issue(0, 0, sem_a)

      def _pair(kk, _):
        k0 = 2 * kk
        k1 = k0 + 1

        @pl.when(k1 < nch)
        def _issue_odd():
          _issue(k1, CH2, sem_b)

        _wait(k0, 0, sem_a)
        _do_chunk(k0 * CH2, 0)

        @pl.when(k1 < nch)
        def _odd_chunk():
          @pl.when(k1 + 1 < nch)
          def _issue_next_even():
            _issue(k1 + 1, 0, sem_a)

          _wait(k1, CH2, sem_b)
          _do_chunk(k1 * CH2, CH2)

        return 0

      lax.fori_loop(0, (nch + 1) >> 1, _pair, 0)
      return 0

    lax.fori_loop(0, NW, _src, 0)

    for q in range(BKT // CH2):
      def _norm(i, _):
        fb = (q * CH2 + i) * STR
        rs16 = acc_v[pl.ds(fb + D, 16)]
        inv16 = 1.0 / (rs16 + 1e-9)
        inv = inv16[0]
        for j0 in range(0, D // 16, 8):
          vs = [acc_v[pl.ds(fb + (j0 + j) * 16, 16)] * inv for j in range(8)]
          for j in range(8):
            gbuf[i, pl.ds((j0 + j) * 16, 16)] = vs[j]
        return 0

      lax.fori_loop(0, CH2, _norm, 0)
      pltpu.sync_copy(gbuf.at[pl.ds(0, CH2)],
                      accn_out.at[pl.ds(w * BKT + q * CH2, CH2)])

  return _p2


def _mm_body(x_ref, w_ref, o_ref):
    o_ref[...] = jnp.dot(x_ref[...], w_ref[...],
                         preferred_element_type=jnp.float32)


def _epi_body(acc_ref, b0_ref, fc_ref, bf_ref, o_ref):
    vh = acc_ref[...] + b0_ref[...]
    t = jnp.sum(vh * fc_ref[...], axis=1, keepdims=True) + bf_ref[...]
    g = jax.nn.sigmoid(t)
    zero = jnp.zeros_like(vh)
    o_ref[...] = jnp.where(vh < 0, zero, vh) + g * jnp.where(vh > 0, zero, vh)


def kernel(x, edge_index, edge_attr, W0, b0, fc0, bf0):
    h = pl.pallas_call(
        _mm_body,
        grid=(10,),
        in_specs=[
            pl.BlockSpec((N // 10, D_IN), lambda i: (i, 0)),
            pl.BlockSpec((D_IN, D), lambda i: (0, 0)),
        ],
        out_specs=pl.BlockSpec((N // 10, D), lambda i: (i, 0)),
        out_shape=jax.ShapeDtypeStruct((N, D), jnp.float32),
    )(x, W0)

    row = edge_index[0]
    col = edge_index[1]

    pk_a, attr_a, counts = _build_p1()(row, col, edge_attr)

    accn = _build_p2()(h, pk_a, attr_a, counts)

    out = pl.pallas_call(
        _epi_body,
        grid=(10,),
        in_specs=[
            pl.BlockSpec((N // 10, D), lambda i: (i, 0)),
            pl.BlockSpec((1, D), lambda i: (0, 0)),
            pl.BlockSpec((1, D), lambda i: (0, 0)),
            pl.BlockSpec((1, 1), lambda i: (0, 0)),
        ],
        out_specs=pl.BlockSpec((N // 10, D), lambda i: (i, 0)),
        out_shape=jax.ShapeDtypeStruct((N, D), jnp.float32),
    )(accn, b0.reshape(1, D), fc0.reshape(1, D), bf0.reshape(1, 1))
    return out


# speedup vs baseline: 1.7751x; 1.2845x over previous
import functools

import jax
import jax.numpy as jnp
from jax import lax
from jax.experimental import pallas as pl
from jax.experimental.pallas import tpu as pltpu
from jax.experimental.pallas import tpu_sc as plsc

N = 10000
D_IN = 128
D = 256
E = 320000

NC = 2
NS = 16
NW = NC * NS
BKT = 320
TRASH = BKT
STR = D + 16
CAP = 480
NSEG = NW * NW * CAP

E_T1 = E // NW
SC1 = 2000
NSC1 = E_T1 // SC1
CH = 80
NCH1 = SC1 // CH

MGC = 52429
SHF = 24


@functools.cache
def _build_p1():
  mesh = plsc.VectorSubcoreMesh(
      core_axis_name="c", subcore_axis_name="s", num_cores=NC, num_subcores=NS)

  @functools.partial(
      pl.kernel,
      out_type=(
          jax.ShapeDtypeStruct((NSEG,), jnp.int32),
          jax.ShapeDtypeStruct((NSEG,), jnp.float32),
          jax.ShapeDtypeStruct((NW * NW,), jnp.int32),
      ),
      mesh=mesh,
      scratch_types=[
          pltpu.VMEM((SC1,), jnp.int32),
          pltpu.VMEM((SC1,), jnp.int32),
          pltpu.VMEM((SC1,), jnp.float32),
          pltpu.VMEM((SC1,), jnp.int32),
          pltpu.VMEM((NCH1, CH), jnp.int32),
          pltpu.VMEM((NW,), jnp.int32),
          pltpu.SMEM((NW,), jnp.int32),
          pltpu.SemaphoreType.DMA,
      ],
  )
  def _p1(row_hbm, col_hbm, attr_hbm, pk_out, attr_out, cnt_out,
          row_v, col_v, attr_v, locbuf, pos2d, cntbuf, ctr_s, sem):
    c = lax.axis_index("c")
    s = lax.axis_index("s")
    w = s * NC + c
    e0 = w * E_T1
    iota = lax.iota(jnp.int32, 16)
    for b in range(NW):
      ctr_s[b] = 0

    def _super(m, _):
      es = e0 + m * SC1
      pltpu.sync_copy(row_hbm.at[pl.ds(es, SC1)], row_v)
      pltpu.sync_copy(col_hbm.at[pl.ds(es, SC1)], col_v)
      pltpu.sync_copy(attr_hbm.at[pl.ds(es, SC1)], attr_v)

      def _chunk(k, _):
        cb = k * CH

        @pl.when(k > 0)
        def _drain_prev():
          cbp = cb - CH
          idxp = pos2d.at[k - 1]
          pltpu.make_async_copy(
              locbuf.at[pl.ds(cbp, CH)], pk_out.at[idxp], sem).wait()
          pltpu.make_async_copy(
              attr_v.at[pl.ds(cbp, CH)], attr_out.at[idxp], sem).wait()

        for j in range(CH // 16):
          r16 = row_v[pl.ds(cb + j * 16, 16)]
          c16 = col_v[pl.ds(cb + j * 16, 16)]
          b16 = (r16 * MGC) >> SHF
          loc16 = r16 - b16 * BKT
          pos16 = iota * 0
          for l in range(16):
            b = b16[l]
            p = jnp.minimum(ctr_s[b], CAP - 1)
            ctr_s[b] = p + 1
            pos16 = jnp.where(iota == l, (b * NW + w) * CAP + p, pos16)
          locbuf[pl.ds(cb + j * 16, 16)] = loc16 * 16384 + c16
          pos2d[k, pl.ds(j * 16, 16)] = pos16
        idx = pos2d.at[k]
        pltpu.async_copy(locbuf.at[pl.ds(cb, CH)], pk_out.at[idx], sem)
        pltpu.async_copy(attr_v.at[pl.ds(cb, CH)], attr_out.at[idx], sem)
        return 0

      lax.fori_loop(0, NCH1, _chunk, 0)
      cbl = SC1 - CH
      idxl = pos2d.at[NCH1 - 1]
      pltpu.make_async_copy(
          locbuf.at[pl.ds(cbl, CH)], pk_out.at[idxl], sem).wait()
      pltpu.make_async_copy(
          attr_v.at[pl.ds(cbl, CH)], attr_out.at[idxl], sem).wait()
      return 0

    lax.fori_loop(0, NSC1, _super, 0)

    v0 = iota * 0
    v1 = iota * 0
    for l in range(16):
      v0 = jnp.where(iota == l, ctr_s[l], v0)
      v1 = jnp.where(iota == l, ctr_s[16 + l], v1)
    cntbuf[pl.ds(0, 16)] = v0
    cntbuf[pl.ds(16, 16)] = v1
    pltpu.sync_copy(cntbuf, cnt_out.at[pl.ds(w * NW, NW)])

  return _p1


ACC_W = (BKT + 8) * STR
NZACC = ACC_W // 16
CH2 = 64
CH4 = 32


@functools.cache
def _build_p2():
  mesh = plsc.VectorSubcoreMesh(
      core_axis_name="c", subcore_axis_name="s", num_cores=NC, num_subcores=NS)

  @functools.partial(
      pl.kernel,
      out_type=jax.ShapeDtypeStruct((NW * BKT, D), jnp.float32),
      mesh=mesh,
      scratch_types=[
          pltpu.VMEM((ACC_W,), jnp.float32),
          pltpu.VMEM((2 * CH2, D), jnp.float32),
          pltpu.VMEM((CAP,), jnp.int32),
          pltpu.VMEM((CAP,), jnp.int32),
          pltpu.VMEM((CAP,), jnp.float32),
          pltpu.VMEM((NW * NW,), jnp.int32),
          pltpu.VMEM((CH4,), jnp.int32),
          pltpu.VMEM((CH4,), jnp.int32),
          pltpu.VMEM((CH4,), jnp.int32),
          pltpu.VMEM((CH4,), jnp.int32),
          pltpu.SemaphoreType.DMA,
          pltpu.SemaphoreType.DMA,
          pltpu.SemaphoreType.DMA,
          pltpu.SemaphoreType.DMA,
          pltpu.SemaphoreType.DMA,
      ],
  )
  def _p2(h_hbm, pk_hbm, attr_hbm, cnt_hbm, accn_out,
          acc_v, gbuf, segl, segc, sega, cnt_v, idxc_0, idxc_1, idxc_2,
          idxc_3, sem_0, sem_1, sem_2, sem_3, sem_s):
    c = lax.axis_index("c")
    s = lax.axis_index("s")
    w = s * NC + c
    iota = lax.iota(jnp.int32, 16)
    zero16 = jnp.zeros((16,), jnp.float32)
    fzero = jnp.zeros((16,), jnp.float32)

    def _zacc(i, _):
      acc_v[pl.ds(i * 16, 16)] = zero16
      return 0

    lax.fori_loop(0, NZACC, _zacc, 0)
    pltpu.sync_copy(cnt_hbm, cnt_v)

    def _do_chunk(cb, h0):
      for jj in range(CH4 // 16):
        l16 = segl[pl.ds(cb + jj * 16, 16)]
        a16 = sega[pl.ds(cb + jj * 16, 16)]
        for l in range(16):
          loc = l16[l]
          a = a16[l]
          fb = loc * STR
          gi = h0 + jj * 16 + l
          for j0 in range(0, D // 16, 8):
            vs = [gbuf[gi, pl.ds((j0 + j) * 16, 16)] * a for j in range(8)]
            for j in range(8):
              plsc.addupdate(acc_v.at[pl.ds(fb + (j0 + j) * 16, 16)], vs[j])
          av = jnp.where(iota == 0, a, fzero)
          plsc.addupdate(acc_v.at[pl.ds(fb + D, 16)], av)

    def _issue(k, h0, idxc, sem):
      for j in range(CH4 // 16):
        idxc[pl.ds(j * 16, 16)] = segc[pl.ds(k * CH4 + j * 16, 16)]
      pltpu.async_copy(h_hbm.at[idxc], gbuf.at[pl.ds(h0, CH4)], sem)

    def _wait(h0, idxc, sem):
      pltpu.make_async_copy(h_hbm.at[idxc],
                            gbuf.at[pl.ds(h0, CH4)], sem).wait()

    def _src(src, _):
      off = src * NW + (w & 16)
      vv = cnt_v[pl.ds(off, 16)]
      iw = (iota + (w & 15)) & 15
      g = lax.gather(
          vv, iw[:, None],
          lax.GatherDimensionNumbers(offset_dims=(), collapsed_slice_dims=(0,),
                                     start_index_map=(0,)),
          (1,), mode=lax.GatherScatterMode.PROMISE_IN_BOUNDS)
      cnt = g[0]
      segbase = (w * NW + src) * CAP
      d1 = pltpu.async_copy(pk_hbm.at[pl.ds(segbase, CAP)], segl, sem_s)
      d2 = pltpu.async_copy(attr_hbm.at[pl.ds(segbase, CAP)], sega, sem_s)
      d1.wait()
      d2.wait()
      for jj in range(CAP // 16):
        sl = pl.ds(jj * 16, 16)
        valid = (jj * 16 + iota) < cnt
        pk = jnp.where(valid, segl[sl], TRASH * 16384)
        segc[sl] = pk & 16383
        segl[sl] = lax.shift_right_logical(pk, 14)
      nch = (cnt + (CH4 - 1)) >> 5
      bufs = ((idxc_0, sem_0), (idxc_1, sem_1),
              (idxc_2, sem_2), (idxc_3, sem_3))

      for q in range(4):
        @pl.when(q < nch)
        def _prime(q=q):
          _issue(q, q * CH4, bufs[q][0], bufs[q][1])

      def _quad(kk, _):
        for q in range(4):
          k = 4 * kk + q

          @pl.when(k < nch)
          def _one(k=k, q=q):
            _wait(q * CH4, bufs[q][0], bufs[q][1])
            _do_chunk(k * CH4, q * CH4)

            @pl.when(k + 4 < nch)
            def _refill():
              _issue(k + 4, q * CH4, bufs[q][0], bufs[q][1])

        return 0

      lax.fori_loop(0, (nch + 3) >> 2, _quad, 0)
      return 0

    lax.fori_loop(0, NW, _src, 0)

    for q in range(BKT // CH2):
      def _norm(i, _):
        fb = (q * CH2 + i) * STR
        rs16 = acc_v[pl.ds(fb + D, 16)]
        inv16 = 1.0 / (rs16 + 1e-9)
        inv = inv16[0]
        for j0 in range(0, D // 16, 8):
          vs = [acc_v[pl.ds(fb + (j0 + j) * 16, 16)] * inv for j in range(8)]
          for j in range(8):
            gbuf[i, pl.ds((j0 + j) * 16, 16)] = vs[j]
        return 0

      lax.fori_loop(0, CH2, _norm, 0)
      pltpu.sync_copy(gbuf.at[pl.ds(0, CH2)],
                      accn_out.at[pl.ds(w * BKT + q * CH2, CH2)])

  return _p2


def _mm_body(x_ref, w_ref, o_ref):
    o_ref[...] = jnp.dot(x_ref[...], w_ref[...],
                         preferred_element_type=jnp.float32)


def _epi_body(acc_ref, b0_ref, fc_ref, bf_ref, o_ref):
    vh = acc_ref[...] + b0_ref[...]
    t = jnp.sum(vh * fc_ref[...], axis=1, keepdims=True) + bf_ref[...]
    g = jax.nn.sigmoid(t)
    zero = jnp.zeros_like(vh)
    o_ref[...] = jnp.where(vh < 0, zero, vh) + g * jnp.where(vh > 0, zero, vh)


def kernel(x, edge_index, edge_attr, W0, b0, fc0, bf0):
    h = pl.pallas_call(
        _mm_body,
        grid=(10,),
        in_specs=[
            pl.BlockSpec((N // 10, D_IN), lambda i: (i, 0)),
            pl.BlockSpec((D_IN, D), lambda i: (0, 0)),
        ],
        out_specs=pl.BlockSpec((N // 10, D), lambda i: (i, 0)),
        out_shape=jax.ShapeDtypeStruct((N, D), jnp.float32),
    )(x, W0)

    row = edge_index[0]
    col = edge_index[1]

    pk_a, attr_a, counts = _build_p1()(row, col, edge_attr)

    accn = _build_p2()(h, pk_a, attr_a, counts)

    out = pl.pallas_call(
        _epi_body,
        grid=(10,),
        in_specs=[
            pl.BlockSpec((N // 10, D), lambda i: (i, 0)),
            pl.BlockSpec((1, D), lambda i: (0, 0)),
            pl.BlockSpec((1, D), lambda i: (0, 0)),
            pl.BlockSpec((1, 1), lambda i: (0, 0)),
        ],
        out_specs=pl.BlockSpec((N // 10, D), lambda i: (i, 0)),
        out_shape=jax.ShapeDtypeStruct((N, D), jnp.float32),
    )(accn, b0.reshape(1, D), fc0.reshape(1, D), bf0.reshape(1, 1))
    return out


# speedup vs baseline: 1.7955x; 1.0115x over previous
import functools

import jax
import jax.numpy as jnp
from jax import lax
from jax.experimental import pallas as pl
from jax.experimental.pallas import tpu as pltpu
from jax.experimental.pallas import tpu_sc as plsc

N = 10000
D_IN = 128
D = 256
E = 320000

NC = 2
NS = 16
NW = NC * NS
BKT = 320
TRASH = BKT
STR = D + 16
CAP = 480
NSEG = NW * NW * CAP

E_T1 = E // NW
SC1 = 2000
NSC1 = E_T1 // SC1
CH = 80
NCH1 = SC1 // CH

MGC = 52429
SHF = 24


@functools.cache
def _build_p1():
  mesh = plsc.VectorSubcoreMesh(
      core_axis_name="c", subcore_axis_name="s", num_cores=NC, num_subcores=NS)

  @functools.partial(
      pl.kernel,
      out_type=(
          jax.ShapeDtypeStruct((NSEG,), jnp.int32),
          jax.ShapeDtypeStruct((NSEG,), jnp.float32),
          jax.ShapeDtypeStruct((NW * NW,), jnp.int32),
      ),
      mesh=mesh,
      scratch_types=[
          pltpu.VMEM((SC1,), jnp.int32),
          pltpu.VMEM((SC1,), jnp.int32),
          pltpu.VMEM((SC1,), jnp.float32),
          pltpu.VMEM((SC1,), jnp.int32),
          pltpu.VMEM((NCH1, CH), jnp.int32),
          pltpu.VMEM((NW,), jnp.int32),
          pltpu.SMEM((NW,), jnp.int32),
          pltpu.SemaphoreType.DMA,
          pltpu.SemaphoreType.DMA,
      ],
  )
  def _p1(row_hbm, col_hbm, attr_hbm, pk_out, attr_out, cnt_out,
          row_v, col_v, attr_v, locbuf, pos2d, cntbuf, ctr_s, sem_p, sem_t):
    c = lax.axis_index("c")
    s = lax.axis_index("s")
    w = s * NC + c
    e0 = w * E_T1
    iota = lax.iota(jnp.int32, 16)
    for b in range(NW):
      ctr_s[b] = 0

    def _super(m, _):
      es = e0 + m * SC1
      pltpu.sync_copy(row_hbm.at[pl.ds(es, SC1)], row_v)
      pltpu.sync_copy(col_hbm.at[pl.ds(es, SC1)], col_v)
      pltpu.sync_copy(attr_hbm.at[pl.ds(es, SC1)], attr_v)

      def _chunk(k, _):
        cb = k * CH

        @pl.when(k > 3)
        def _drain_prev():
          cbp = cb - 4 * CH
          idxp = pos2d.at[k - 4]
          pltpu.make_async_copy(
              locbuf.at[pl.ds(cbp, CH)], pk_out.at[idxp], sem_p).wait()
          pltpu.make_async_copy(
              attr_v.at[pl.ds(cbp, CH)], attr_out.at[idxp], sem_t).wait()

        for j in range(CH // 16):
          r16 = row_v[pl.ds(cb + j * 16, 16)]
          c16 = col_v[pl.ds(cb + j * 16, 16)]
          b16 = (r16 * MGC) >> SHF
          loc16 = r16 - b16 * BKT
          pos16 = iota * 0
          for l in range(16):
            b = b16[l]
            p = jnp.minimum(ctr_s[b], CAP - 1)
            ctr_s[b] = p + 1
            pos16 = jnp.where(iota == l, (b * NW + w) * CAP + p, pos16)
          locbuf[pl.ds(cb + j * 16, 16)] = loc16 * 16384 + c16
          pos2d[k, pl.ds(j * 16, 16)] = pos16
        idx = pos2d.at[k]
        pltpu.async_copy(locbuf.at[pl.ds(cb, CH)], pk_out.at[idx], sem_p)
        pltpu.async_copy(attr_v.at[pl.ds(cb, CH)], attr_out.at[idx], sem_t)
        return 0

      lax.fori_loop(0, NCH1, _chunk, 0)
      for t in range(4):
        cbl = SC1 - (4 - t) * CH
        idxl = pos2d.at[NCH1 - 4 + t]
        pltpu.make_async_copy(
            locbuf.at[pl.ds(cbl, CH)], pk_out.at[idxl], sem_p).wait()
        pltpu.make_async_copy(
            attr_v.at[pl.ds(cbl, CH)], attr_out.at[idxl], sem_t).wait()
      return 0

    lax.fori_loop(0, NSC1, _super, 0)

    v0 = iota * 0
    v1 = iota * 0
    for l in range(16):
      v0 = jnp.where(iota == l, ctr_s[l], v0)
      v1 = jnp.where(iota == l, ctr_s[16 + l], v1)
    cntbuf[pl.ds(0, 16)] = v0
    cntbuf[pl.ds(16, 16)] = v1
    pltpu.sync_copy(cntbuf, cnt_out.at[pl.ds(w * NW, NW)])

  return _p1


ACC_W = (BKT + 8) * STR
NZACC = ACC_W // 16
CH2 = 64
CH4 = 32


@functools.cache
def _build_p2():
  mesh = plsc.VectorSubcoreMesh(
      core_axis_name="c", subcore_axis_name="s", num_cores=NC, num_subcores=NS)

  @functools.partial(
      pl.kernel,
      out_type=jax.ShapeDtypeStruct((NW * BKT, D), jnp.float32),
      mesh=mesh,
      scratch_types=[
          pltpu.VMEM((ACC_W,), jnp.float32),
          pltpu.VMEM((2 * CH2, D), jnp.float32),
          pltpu.VMEM((CAP,), jnp.int32),
          pltpu.VMEM((CAP,), jnp.int32),
          pltpu.VMEM((CAP,), jnp.float32),
          pltpu.VMEM((NW * NW,), jnp.int32),
          pltpu.VMEM((CH4,), jnp.int32),
          pltpu.VMEM((CH4,), jnp.int32),
          pltpu.VMEM((CH4,), jnp.int32),
          pltpu.VMEM((CH4,), jnp.int32),
          pltpu.SemaphoreType.DMA,
          pltpu.SemaphoreType.DMA,
          pltpu.SemaphoreType.DMA,
          pltpu.SemaphoreType.DMA,
          pltpu.SemaphoreType.DMA,
      ],
  )
  def _p2(h_hbm, pk_hbm, attr_hbm, cnt_hbm, accn_out,
          acc_v, gbuf, segl, segc, sega, cnt_v, idxc_0, idxc_1, idxc_2,
          idxc_3, sem_0, sem_1, sem_2, sem_3, sem_s):
    c = lax.axis_index("c")
    s = lax.axis_index("s")
    w = s * NC + c
    iota = lax.iota(jnp.int32, 16)
    zero16 = jnp.zeros((16,), jnp.float32)
    fzero = jnp.zeros((16,), jnp.float32)

    def _zacc(i, _):
      acc_v[pl.ds(i * 16, 16)] = zero16
      return 0

    lax.fori_loop(0, NZACC, _zacc, 0)
    pltpu.sync_copy(cnt_hbm, cnt_v)

    def _do_chunk(cb, h0):
      for jj in range(CH4 // 16):
        l16 = segl[pl.ds(cb + jj * 16, 16)]
        a16 = sega[pl.ds(cb + jj * 16, 16)]
        for l in range(16):
          loc = l16[l]
          a = a16[l]
          fb = loc * STR
          gi = h0 + jj * 16 + l
          for j0 in range(0, D // 16, 8):
            vs = [gbuf[gi, pl.ds((j0 + j) * 16, 16)] * a for j in range(8)]
            for j in range(8):
              plsc.addupdate(acc_v.at[pl.ds(fb + (j0 + j) * 16, 16)], vs[j])
          av = jnp.where(iota == 0, a, fzero)
          plsc.addupdate(acc_v.at[pl.ds(fb + D, 16)], av)

    def _issue(k, h0, idxc, sem):
      for j in range(CH4 // 16):
        idxc[pl.ds(j * 16, 16)] = segc[pl.ds(k * CH4 + j * 16, 16)]
      pltpu.async_copy(h_hbm.at[idxc], gbuf.at[pl.ds(h0, CH4)], sem)

    def _wait(h0, idxc, sem):
      pltpu.make_async_copy(h_hbm.at[idxc],
                            gbuf.at[pl.ds(h0, CH4)], sem).wait()

    def _src(src, _):
      off = src * NW + (w & 16)
      vv = cnt_v[pl.ds(off, 16)]
      iw = (iota + (w & 15)) & 15
      g = lax.gather(
          vv, iw[:, None],
          lax.GatherDimensionNumbers(offset_dims=(), collapsed_slice_dims=(0,),
                                     start_index_map=(0,)),
          (1,), mode=lax.GatherScatterMode.PROMISE_IN_BOUNDS)
      cnt = g[0]
      segbase = (w * NW + src) * CAP
      d1 = pltpu.async_copy(pk_hbm.at[pl.ds(segbase, CAP)], segl, sem_s)
      d2 = pltpu.async_copy(attr_hbm.at[pl.ds(segbase, CAP)], sega, sem_s)
      d1.wait()
      d2.wait()
      for jj in range(CAP // 16):
        sl = pl.ds(jj * 16, 16)
        valid = (jj * 16 + iota) < cnt
        pk = jnp.where(valid, segl[sl], TRASH * 16384)
        segc[sl] = pk & 16383
        segl[sl] = lax.shift_right_logical(pk, 14)
      nch = (cnt + (CH4 - 1)) >> 5
      bufs = ((idxc_0, sem_0), (idxc_1, sem_1),
              (idxc_2, sem_2), (idxc_3, sem_3))

      for q in range(4):
        @pl.when(q < nch)
        def _prime(q=q):
          _issue(q, q * CH4, bufs[q][0], bufs[q][1])

      def _quad(kk, _):
        for q in range(4):
          k = 4 * kk + q

          @pl.when(k < nch)
          def _one(k=k, q=q):
            _wait(q * CH4, bufs[q][0], bufs[q][1])
            _do_chunk(k * CH4, q * CH4)

            @pl.when(k + 4 < nch)
            def _refill():
              _issue(k + 4, q * CH4, bufs[q][0], bufs[q][1])

        return 0

      lax.fori_loop(0, (nch + 3) >> 2, _quad, 0)
      return 0

    lax.fori_loop(0, NW, _src, 0)

    for q in range(BKT // CH2):
      def _norm(i, _):
        fb = (q * CH2 + i) * STR
        rs16 = acc_v[pl.ds(fb + D, 16)]
        inv16 = 1.0 / (rs16 + 1e-9)
        inv = inv16[0]
        for j0 in range(0, D // 16, 8):
          vs = [acc_v[pl.ds(fb + (j0 + j) * 16, 16)] * inv for j in range(8)]
          for j in range(8):
            gbuf[i, pl.ds((j0 + j) * 16, 16)] = vs[j]
        return 0

      lax.fori_loop(0, CH2, _norm, 0)
      pltpu.sync_copy(gbuf.at[pl.ds(0, CH2)],
                      accn_out.at[pl.ds(w * BKT + q * CH2, CH2)])

  return _p2


def _mm_body(x_ref, w_ref, o_ref):
    o_ref[...] = jnp.dot(x_ref[...], w_ref[...],
                         preferred_element_type=jnp.float32)


def _epi_body(acc_ref, b0_ref, fc_ref, bf_ref, o_ref):
    vh = acc_ref[...] + b0_ref[...]
    t = jnp.sum(vh * fc_ref[...], axis=1, keepdims=True) + bf_ref[...]
    g = jax.nn.sigmoid(t)
    zero = jnp.zeros_like(vh)
    o_ref[...] = jnp.where(vh < 0, zero, vh) + g * jnp.where(vh > 0, zero, vh)


def kernel(x, edge_index, edge_attr, W0, b0, fc0, bf0):
    h = pl.pallas_call(
        _mm_body,
        grid=(10,),
        in_specs=[
            pl.BlockSpec((N // 10, D_IN), lambda i: (i, 0)),
            pl.BlockSpec((D_IN, D), lambda i: (0, 0)),
        ],
        out_specs=pl.BlockSpec((N // 10, D), lambda i: (i, 0)),
        out_shape=jax.ShapeDtypeStruct((N, D), jnp.float32),
    )(x, W0)

    row = edge_index[0]
    col = edge_index[1]

    pk_a, attr_a, counts = _build_p1()(row, col, edge_attr)

    accn = _build_p2()(h, pk_a, attr_a, counts)

    out = pl.pallas_call(
        _epi_body,
        grid=(10,),
        in_specs=[
            pl.BlockSpec((N // 10, D), lambda i: (i, 0)),
            pl.BlockSpec((1, D), lambda i: (0, 0)),
            pl.BlockSpec((1, D), lambda i: (0, 0)),
            pl.BlockSpec((1, 1), lambda i: (0, 0)),
        ],
        out_specs=pl.BlockSpec((N // 10, D), lambda i: (i, 0)),
        out_shape=jax.ShapeDtypeStruct((N, D), jnp.float32),
    )(accn, b0.reshape(1, D), fc0.reshape(1, D), bf0.reshape(1, 1))
    return out
